# baseline (speedup 1.0000x reference)
import functools

import jax
import jax.numpy as jnp
from jax import lax
from jax.experimental import pallas as pl
from jax.experimental.pallas import tpu as pltpu
from jax.experimental.pallas import tpu_sc as plsc

D = 128
DH = D // 2
LANES = 16
NC = 2
NS = 16
CH = 128
SUP = 4
NJH = DH // LANES
SCH = 64


def _ceil_to(x, m):
    return (x + m - 1) // m * m


def _tc_pre_body(xq, xt, m, wq, wt, wm, y0, y1, y2, y3):
    f32 = jnp.float32
    y0[...] = jnp.dot(xq[...], wq[...], preferred_element_type=f32)
    y1[...] = jnp.dot(xt[...], wt[...], preferred_element_type=f32)
    y2[...] = lax.dot_general(xt[...], wm[...], (((1,), (1,)), ((), ())),
                              preferred_element_type=f32)
    y3[...] = jnp.dot(xq[...] * m[...], wm[...], preferred_element_type=f32)


def _tc_pre(Xq, Xt, maskf, Wq, Wt, Wm, n):
    bn = 1000
    grid = (n // bn,)
    row_spec = pl.BlockSpec((bn, D), lambda i: (i, 0))
    w_spec = pl.BlockSpec((D, D), lambda i: (0, 0))
    m_spec = pl.BlockSpec((bn, 1), lambda i: (i, 0))
    out = jax.ShapeDtypeStruct((n, D), jnp.float32)
    return pl.pallas_call(
        _tc_pre_body,
        grid=grid,
        in_specs=[row_spec, row_spec, m_spec, w_spec, w_spec, w_spec],
        out_specs=[row_spec, row_spec, row_spec, row_spec],
        out_shape=[out, out, out, out],
    )(Xq, Xt, maskf, Wq, Wt, Wm)


def _make_sc(npad, nsi, nsx):
    mesh = plsc.VectorSubcoreMesh(
        core_axis_name="c", subcore_axis_name="s", num_cores=NC,
        num_subcores=NS)
    ph = npad // 2
    rps = ph // NS
    nz = rps // SCH

    @functools.partial(
        pl.kernel,
        out_type=jax.ShapeDtypeStruct((2, 2, ph, D), jnp.float32),
        mesh=mesh,
        scratch_types=[
            pltpu.VMEM_SHARED((ph, D), jnp.float32),
            pltpu.VMEM_SHARED((ph, D), jnp.float32),
            pltpu.VMEM((SUP, CH), jnp.int32),
            pltpu.VMEM((SUP, CH), jnp.int32),
            pltpu.VMEM((SUP, CH), jnp.float32),
            pltpu.VMEM((SUP, CH), jnp.float32),
            pltpu.VMEM((SUP, CH), jnp.float32),
            pltpu.VMEM((CH, D), jnp.float32),
            pltpu.VMEM((CH, D), jnp.float32),
            pltpu.SemaphoreType.DMA,
            pltpu.SemaphoreType.DMA,
            pltpu.SemaphoreType.DMA,
        ],
    )
    def sc_kernel(tqi0, tqi1, tqc0, tqc1, tti0, tti1, ttc0, ttc1,
                  qi_g, qi_s, qi_p, qi_l, qi_h,
                  qc_g, qc_s, qc_p, qc_l, qc_h,
                  ti_g, ti_s, ti_p, ti_l, ti_h,
                  tc_g, tc_s, tc_p, tc_l, tc_h,
                  out,
                  tbl, acc, gb, sb, pb, nlb, nhb, rows0, rows1,
                  semg0, semg1, semi):
        s = lax.axis_index("s")
        rows = (rows0, rows1)
        semg = (semg0, semg1)
        zv = jnp.zeros((LANES,), jnp.float32)

        def run_edges(streams, nsup):
            e_g, e_s, e_p, e_l, e_h = streams
            base = s * nsup * SUP

            def body(i, cc):
                rb = base + i * SUP
                di = [pltpu.async_copy(r.at[pl.ds(rb, SUP)], b, semi)
                      for r, b in ((e_g, gb), (e_s, sb), (e_p, pb),
                                   (e_l, nlb), (e_h, nhb))]
                for d in di:
                    d.wait()
                dg = [None, None]
                dg[0] = pltpu.async_copy(tbl.at[gb.at[0]], rows[0], semg[0])
                for k in range(SUP):
                    p = k & 1
                    if k + 1 < SUP:
                        dg[1 - p] = pltpu.async_copy(
                            tbl.at[gb.at[k + 1]], rows[1 - p], semg[1 - p])
                    dg[p].wait()
                    buf = rows[p]

                    def repack(e16, c2):
                        spv = pb[k, pl.ds(e16 * LANES, LANES)]
                        nlv = nlb[k, pl.ds(e16 * LANES, LANES)]
                        nhv = nhb[k, pl.ds(e16 * LANES, LANES)]
                        for l in range(LANES):
                            spe = jnp.full((LANES,), spv[l])
                            nle = jnp.full((LANES,), nlv[l])
                            nhe = jnp.full((LANES,), nhv[l])
                            e = e16 * LANES + l
                            for j in range(NJH):
                                slo = pl.ds(j * LANES, LANES)
                                shi = pl.ds(DH + j * LANES, LANES)
                                a = buf[e, slo]
                                sel = a + (buf[e, shi] - a) * spe
                                buf[e, slo] = sel * nle
                                buf[e, shi] = sel * nhe
                        return c2

                    lax.fori_loop(0, CH // LANES, repack, 0)
                    pltpu.sync_copy(buf, acc.at[sb.at[k]], add=True)
                return cc

            lax.fori_loop(0, nsup, body, 0)

        def stage(src_hbm):
            for k in range(nz):
                off = s * rps + k * SCH
                pltpu.sync_copy(src_hbm.at[pl.ds(off, SCH)],
                                rows0.at[pl.ds(0, SCH)])
                pltpu.sync_copy(rows0.at[pl.ds(0, SCH)],
                                tbl.at[pl.ds(off, SCH)])

        def run_graph(g, tbl_i, str_i, tbl_c, str_c):
            for h in range(2):
                def zr(e, cc):
                    for j in range(D // LANES):
                        rows1[e, pl.ds(j * LANES, LANES)] = zv
                    return cc

                lax.fori_loop(0, SCH, zr, 0)
                for k in range(nz):
                    pltpu.sync_copy(
                        rows1.at[pl.ds(0, SCH)],
                        acc.at[pl.ds(s * rps + k * SCH, SCH)])
                stage(tbl_i[h])
                plsc.subcore_barrier()
                run_edges(str_i, nsi)
                plsc.subcore_barrier()
                stage(tbl_c[h])
                plsc.subcore_barrier()
                run_edges(str_c, nsx)
                plsc.subcore_barrier()
                for k in range(nz):
                    off = s * rps + k * SCH
                    pltpu.sync_copy(acc.at[pl.ds(off, SCH)],
                                    rows0.at[pl.ds(0, SCH)])

                    def rel(e, cc):
                        for j in range(D // LANES):
                            sl = pl.ds(j * LANES, LANES)
                            rows0[e, sl] = jnp.maximum(rows0[e, sl], 0.0)
                        return cc

                    lax.fori_loop(0, SCH, rel, 0)
                    pltpu.sync_copy(rows0.at[pl.ds(0, SCH)],
                                    out.at[g, h, pl.ds(off, SCH)])
                plsc.subcore_barrier()

        c = lax.axis_index("c")

        @pl.when(c == 0)
        def _():
            run_graph(0, (tqi0, tqi1), (qi_g, qi_s, qi_p, qi_l, qi_h),
                      (tqc0, tqc1), (qc_g, qc_s, qc_p, qc_l, qc_h))

        @pl.when(c == 1)
        def _():
            run_graph(1, (tti0, tti1), (ti_g, ti_s, ti_p, ti_l, ti_h),
                      (ttc0, ttc1), (tc_g, tc_s, tc_p, tc_l, tc_h))

    return sc_kernel


def _pad2d(a, total, val):
    e = a.shape[0]
    if e != total:
        a = jnp.concatenate([a, jnp.full((total - e,), val, a.dtype)])
    return a.reshape(-1, CH)


def kernel(Xq, edge_indexq, Xt, edge_indext, norm_q, norm_t, u2v_li,
           node_mask, only_inter, Wq, Wt, Wm):
    n = Xq.shape[0]
    npad = _ceil_to(n, NS * CH)
    dump = n + 8

    maskf = node_mask.astype(jnp.float32)[:, None]
    y_qi, y_ti, y_tc, y_qc = _tc_pre(Xq, Xt, maskf, Wq, Wt, Wm, n)

    def _packed(y):
        yp = jnp.pad(y, ((0, npad - n), (0, 0)))
        return (yp[:, :DH].reshape(npad // 2, D),
                yp[:, DH:].reshape(npad // 2, D))

    tqi = _packed(y_qi)
    tti = _packed(y_ti)
    tqc = _packed(y_tc)
    ttc = _packed(y_qc)

    intra_scale = jnp.where(jnp.asarray(only_inter) != 0, 0.0, 1.0)

    unit = NS * SUP * CH

    def _streams(src, dst, w, total):
        dpar = dst & 1
        return (_pad2d(src >> 1, total, 0),
                _pad2d(dst >> 1, total, dump >> 1),
                _pad2d((src & 1).astype(jnp.float32), total, 0.0),
                _pad2d(w * (1 - dpar).astype(w.dtype), total, 0.0),
                _pad2d(w * dpar.astype(w.dtype), total, 0.0))

    eq = edge_indexq.shape[1]
    et = edge_indext.shape[1]
    ex = u2v_li.shape[1]
    epq = _ceil_to(eq, unit)
    ept = _ceil_to(et, unit)
    epx = _ceil_to(ex, unit)
    assert ept == epq

    ones_x = jnp.ones((ex,), jnp.float32)
    str_qi = _streams(edge_indexq[0], edge_indexq[1],
                      norm_q * intra_scale, epq)
    str_ti = _streams(edge_indext[0], edge_indext[1],
                      norm_t * intra_scale, ept)
    u = u2v_li[0]
    v = u2v_li[1]
    str_qc = _streams(v, u, ones_x, epx)
    str_tc = _streams(u, v, ones_x, epx)

    sc = _make_sc(npad, epq // unit, epx // unit)
    O = sc(tqi[0], tqi[1], tqc[0], tqc[1], tti[0], tti[1], ttc[0], ttc[1],
           *str_qi, *str_qc, *str_ti, *str_tc)
    Oq = jnp.concatenate([O[0, 0].reshape(npad, DH),
                          O[0, 1].reshape(npad, DH)], axis=1)
    Ot = jnp.concatenate([O[1, 0].reshape(npad, DH),
                          O[1, 1].reshape(npad, DH)], axis=1)
    return (Oq[:n], Ot[:n])


# speedup vs baseline: 1.1011x; 1.1011x over previous
import functools

import jax
import jax.numpy as jnp
from jax import lax
from jax.experimental import pallas as pl
from jax.experimental.pallas import tpu as pltpu
from jax.experimental.pallas import tpu_sc as plsc

D = 128
DH = D // 2
LANES = 16
NC = 2
NS = 16
CH = 128
SUP = 4
NJH = DH // LANES
SCH = 64


def _ceil_to(x, m):
    return (x + m - 1) // m * m


def _tc_pre_body(xq, xt, m, wq, wt, wm, y0, y1, y2, y3):
    f32 = jnp.float32
    y0[...] = jnp.dot(xq[...], wq[...], preferred_element_type=f32)
    y1[...] = jnp.dot(xt[...], wt[...], preferred_element_type=f32)
    y2[...] = lax.dot_general(xt[...], wm[...], (((1,), (1,)), ((), ())),
                              preferred_element_type=f32)
    y3[...] = jnp.dot(xq[...] * m[...], wm[...], preferred_element_type=f32)


def _tc_pre(Xq, Xt, maskf, Wq, Wt, Wm, n):
    bn = 1000
    grid = (n // bn,)
    row_spec = pl.BlockSpec((bn, D), lambda i: (i, 0))
    w_spec = pl.BlockSpec((D, D), lambda i: (0, 0))
    m_spec = pl.BlockSpec((bn, 1), lambda i: (i, 0))
    out = jax.ShapeDtypeStruct((n, D), jnp.float32)
    return pl.pallas_call(
        _tc_pre_body,
        grid=grid,
        in_specs=[row_spec, row_spec, m_spec, w_spec, w_spec, w_spec],
        out_specs=[row_spec, row_spec, row_spec, row_spec],
        out_shape=[out, out, out, out],
    )(Xq, Xt, maskf, Wq, Wt, Wm)


def _make_sc(npad, nsi, nsx):
    mesh = plsc.VectorSubcoreMesh(
        core_axis_name="c", subcore_axis_name="s", num_cores=NC,
        num_subcores=NS)
    ph = npad // 2
    rps = ph // NS
    nz = rps // SCH

    @functools.partial(
        pl.kernel,
        out_type=jax.ShapeDtypeStruct((2, 2, ph, D), jnp.float32),
        mesh=mesh,
        scratch_types=[
            pltpu.VMEM_SHARED((ph, D), jnp.float32),
            pltpu.VMEM_SHARED((ph, D), jnp.float32),
            pltpu.VMEM((2, SUP, CH), jnp.int32),
            pltpu.VMEM((2, SUP, CH), jnp.int32),
            pltpu.VMEM((2, SUP, CH), jnp.float32),
            pltpu.VMEM((2, SUP, CH), jnp.float32),
            pltpu.VMEM((2, SUP, CH), jnp.float32),
            pltpu.VMEM((CH, D), jnp.float32),
            pltpu.VMEM((CH, D), jnp.float32),
            pltpu.SemaphoreType.DMA,
            pltpu.SemaphoreType.DMA,
            pltpu.SemaphoreType.DMA,
        ],
    )
    def sc_kernel(tqi0, tqi1, tqc0, tqc1, tti0, tti1, ttc0, ttc1,
                  qi_g, qi_s, qi_p, qi_l, qi_h,
                  qc_g, qc_s, qc_p, qc_l, qc_h,
                  ti_g, ti_s, ti_p, ti_l, ti_h,
                  tc_g, tc_s, tc_p, tc_l, tc_h,
                  out,
                  tbl, acc, gb, sb, pb, nlb, nhb, rows0, rows1,
                  semg0, semg1, semi):
        s = lax.axis_index("s")
        rows = (rows0, rows1)
        semg = (semg0, semg1)
        zv = jnp.zeros((LANES,), jnp.float32)

        def run_edges(streams, nsup):
            pairs = tuple(zip(streams, (gb, sb, pb, nlb, nhb)))
            base = s * nsup * SUP

            def idx_load(half, rb):
                return [pltpu.async_copy(r.at[pl.ds(rb, SUP)], b.at[half],
                                         semi) for r, b in pairs]

            for d in idx_load(0, base):
                d.wait()

            def body(i, cc):
                half = i & 1
                dpre = idx_load(1 - half, base + (i + 1) * SUP)
                dg = [None, None]
                dg[0] = pltpu.async_copy(
                    tbl.at[gb.at[half, 0]], rows[0], semg[0])
                for k in range(SUP):
                    p = k & 1
                    if k + 1 < SUP:
                        dg[1 - p] = pltpu.async_copy(
                            tbl.at[gb.at[half, k + 1]], rows[1 - p],
                            semg[1 - p])
                    dg[p].wait()
                    buf = rows[p]

                    def repack(e16, c2):
                        spv = pb[half, k, pl.ds(e16 * LANES, LANES)]
                        nlv = nlb[half, k, pl.ds(e16 * LANES, LANES)]
                        nhv = nhb[half, k, pl.ds(e16 * LANES, LANES)]
                        for l in range(LANES):
                            spe = jnp.full((LANES,), spv[l])
                            nle = jnp.full((LANES,), nlv[l])
                            nhe = jnp.full((LANES,), nhv[l])
                            e = e16 * LANES + l
                            for j in range(NJH):
                                slo = pl.ds(j * LANES, LANES)
                                shi = pl.ds(DH + j * LANES, LANES)
                                a = buf[e, slo]
                                sel = a + (buf[e, shi] - a) * spe
                                buf[e, slo] = sel * nle
                                buf[e, shi] = sel * nhe
                        return c2

                    lax.fori_loop(0, CH // LANES, repack, 0)
                    pltpu.sync_copy(buf, acc.at[sb.at[half, k]], add=True)
                for d in dpre:
                    d.wait()
                return cc

            lax.fori_loop(0, nsup, body, 0)

        def stage(src_hbm):
            for k in range(nz):
                off = s * rps + k * SCH
                pltpu.sync_copy(src_hbm.at[pl.ds(off, SCH)],
                                rows0.at[pl.ds(0, SCH)])
                pltpu.sync_copy(rows0.at[pl.ds(0, SCH)],
                                tbl.at[pl.ds(off, SCH)])

        def run_graph(g, tbl_i, str_i, tbl_c, str_c):
            for h in range(2):
                def zr(e, cc):
                    for j in range(D // LANES):
                        rows1[e, pl.ds(j * LANES, LANES)] = zv
                    return cc

                lax.fori_loop(0, SCH, zr, 0)
                for k in range(nz):
                    pltpu.sync_copy(
                        rows1.at[pl.ds(0, SCH)],
                        acc.at[pl.ds(s * rps + k * SCH, SCH)])
                stage(tbl_i[h])
                plsc.subcore_barrier()
                run_edges(str_i, nsi)
                plsc.subcore_barrier()
                stage(tbl_c[h])
                plsc.subcore_barrier()
                run_edges(str_c, nsx)
                plsc.subcore_barrier()
                for k in range(nz):
                    off = s * rps + k * SCH
                    pltpu.sync_copy(acc.at[pl.ds(off, SCH)],
                                    rows0.at[pl.ds(0, SCH)])

                    def rel(e, cc):
                        for j in range(D // LANES):
                            sl = pl.ds(j * LANES, LANES)
                            rows0[e, sl] = jnp.maximum(rows0[e, sl], 0.0)
                        return cc

                    lax.fori_loop(0, SCH, rel, 0)
                    pltpu.sync_copy(rows0.at[pl.ds(0, SCH)],
                                    out.at[g, h, pl.ds(off, SCH)])
                plsc.subcore_barrier()

        c = lax.axis_index("c")

        @pl.when(c == 0)
        def _():
            run_graph(0, (tqi0, tqi1), (qi_g, qi_s, qi_p, qi_l, qi_h),
                      (tqc0, tqc1), (qc_g, qc_s, qc_p, qc_l, qc_h))

        @pl.when(c == 1)
        def _():
            run_graph(1, (tti0, tti1), (ti_g, ti_s, ti_p, ti_l, ti_h),
                      (ttc0, ttc1), (tc_g, tc_s, tc_p, tc_l, tc_h))

    return sc_kernel


def _pad2d(a, total, val):
    e = a.shape[0]
    if e != total:
        a = jnp.concatenate([a, jnp.full((total - e,), val, a.dtype)])
    return a.reshape(-1, CH)


def kernel(Xq, edge_indexq, Xt, edge_indext, norm_q, norm_t, u2v_li,
           node_mask, only_inter, Wq, Wt, Wm):
    n = Xq.shape[0]
    npad = _ceil_to(n, NS * CH)
    dump = n + 8

    maskf = node_mask.astype(jnp.float32)[:, None]
    y_qi, y_ti, y_tc, y_qc = _tc_pre(Xq, Xt, maskf, Wq, Wt, Wm, n)

    def _packed(y):
        yp = jnp.pad(y, ((0, npad - n), (0, 0)))
        return (yp[:, :DH].reshape(npad // 2, D),
                yp[:, DH:].reshape(npad // 2, D))

    tqi = _packed(y_qi)
    tti = _packed(y_ti)
    tqc = _packed(y_tc)
    ttc = _packed(y_qc)

    intra_scale = jnp.where(jnp.asarray(only_inter) != 0, 0.0, 1.0)

    unit = NS * SUP * CH

    guard = SUP * CH

    def _streams(src, dst, w, total):
        dpar = dst & 1
        total += guard
        return (_pad2d(src >> 1, total, 0),
                _pad2d(dst >> 1, total, dump >> 1),
                _pad2d((src & 1).astype(jnp.float32), total, 0.0),
                _pad2d(w * (1 - dpar).astype(w.dtype), total, 0.0),
                _pad2d(w * dpar.astype(w.dtype), total, 0.0))

    eq = edge_indexq.shape[1]
    et = edge_indext.shape[1]
    ex = u2v_li.shape[1]
    epq = _ceil_to(eq, unit)
    ept = _ceil_to(et, unit)
    epx = _ceil_to(ex, unit)
    assert ept == epq

    ones_x = jnp.ones((ex,), jnp.float32)
    str_qi = _streams(edge_indexq[0], edge_indexq[1],
                      norm_q * intra_scale, epq)
    str_ti = _streams(edge_indext[0], edge_indext[1],
                      norm_t * intra_scale, ept)
    u = u2v_li[0]
    v = u2v_li[1]
    str_qc = _streams(v, u, ones_x, epx)
    str_tc = _streams(u, v, ones_x, epx)

    sc = _make_sc(npad, epq // unit, epx // unit)
    O = sc(tqi[0], tqi[1], tqc[0], tqc[1], tti[0], tti[1], ttc[0], ttc[1],
           *str_qi, *str_qc, *str_ti, *str_tc)
    Oq = jnp.concatenate([O[0, 0].reshape(npad, DH),
                          O[0, 1].reshape(npad, DH)], axis=1)
    Ot = jnp.concatenate([O[1, 0].reshape(npad, DH),
                          O[1, 1].reshape(npad, DH)], axis=1)
    return (Oq[:n], Ot[:n])
